# initial kernel scaffold (unmeasured)
import jax
import jax.numpy as jnp
from jax import lax
from jax.experimental import pallas as pl
from jax.experimental.pallas import tpu as pltpu

N_DEV = 16


def kernel(x, w_mat):
    m_total, k_per = x.shape
    _, n_cols = w_mat.shape
    m_per = m_total // N_DEV

    def body(x_ref, w_ref, out_ref,
             recv_buf, send_buf, maxes_ref,
             send_sems, recv_sems, credit_sems,
             b_send_sems, b_recv_sems):
        me = lax.axis_index("i")
        left = (me + N_DEV - 1) % N_DEV
        right = (me + 1) % N_DEV

        barrier_sem = pltpu.get_barrier_semaphore()
        for nbr in (left, right):
            pl.semaphore_signal(barrier_sem, inc=1, device_id=(nbr,),
                                device_id_type=pl.DeviceIdType.MESH)
        pl.semaphore_wait(barrier_sem, 2)

        def ring_desc(src_slot, dst_slot, sem_slot, target):
            return pltpu.make_async_remote_copy(
                src_ref=send_buf.at[src_slot],
                dst_ref=recv_buf.at[dst_slot],
                send_sem=send_sems.at[sem_slot],
                recv_sem=recv_sems.at[dst_slot],
                device_id=(target,),
                device_id_type=pl.DeviceIdType.MESH,
            )

        send_desc = [None, None]
        total = None

        for s in range(N_DEV):
            c = (me + 2 * N_DEV - 1 - s) % N_DEV
            k = s % 2
            xs = x_ref[pl.ds(c * m_per, m_per), :]
            p = jnp.dot(xs, w_ref[...], preferred_element_type=jnp.float32)

            if s == 0:
                send_buf[k, :, :] = p
            else:
                j = (s - 1) % 2
                ring_desc(j, j, j, left).wait_recv()
                total = recv_buf[j, :, :] + p
                if s < N_DEV - 1:
                    if send_desc[k] is not None:
                        send_desc[k].wait_send()
                        send_desc[k] = None
                    send_buf[k, :, :] = total
                if s <= N_DEV - 3:
                    pl.semaphore_signal(credit_sems.at[j], inc=1,
                                        device_id=(left,),
                                        device_id_type=pl.DeviceIdType.MESH)

            if s < N_DEV - 1:
                if s >= 2:
                    pl.semaphore_wait(credit_sems.at[k], 1)
                d = ring_desc(k, k, k, right)
                d.start()
                send_desc[k] = d

        for k in range(2):
            if send_desc[k] is not None:
                send_desc[k].wait_send()

        y = jnp.maximum(total, 0.0)
        out_ref[...] = y

        local_max = jnp.max(y)
        maxes_ref[pl.ds(me, 1)] = jnp.full((1, 8, 128), local_max,
                                           dtype=jnp.float32)

        bdescs = []
        for off in range(1, N_DEV):
            t = (me + off) % N_DEV
            bd = pltpu.make_async_remote_copy(
                src_ref=maxes_ref.at[me],
                dst_ref=maxes_ref.at[me],
                send_sem=b_send_sems.at[off - 1],
                recv_sem=b_recv_sems.at[me],
                device_id=(t,),
                device_id_type=pl.DeviceIdType.MESH,
            )
            bd.start()
            bdescs.append(bd)

        for j in range(N_DEV):
            @pl.when(j != me)
            def _(j=j):
                rd = pltpu.make_async_remote_copy(
                    src_ref=maxes_ref.at[j],
                    dst_ref=maxes_ref.at[j],
                    send_sem=b_send_sems.at[0],
                    recv_sem=b_recv_sems.at[j],
                    device_id=(left,),
                    device_id_type=pl.DeviceIdType.MESH,
                )
                rd.wait_recv()

        for bd in bdescs:
            bd.wait_send()

        gmax = jnp.max(maxes_ref[...])
        scale = jnp.maximum(gmax, 1e-30) / 127.0
        q = jnp.clip(jnp.round(out_ref[...] / scale), 0.0, 127.0)
        out_ref[...] = q * scale

    return pl.pallas_call(
        body,
        out_shape=jax.ShapeDtypeStruct((m_per, n_cols), jnp.float32),
        in_specs=[
            pl.BlockSpec(memory_space=pltpu.VMEM),
            pl.BlockSpec(memory_space=pltpu.VMEM),
        ],
        out_specs=pl.BlockSpec(memory_space=pltpu.VMEM),
        scratch_shapes=[
            pltpu.VMEM((2, m_per, n_cols), jnp.float32),
            pltpu.VMEM((2, m_per, n_cols), jnp.float32),
            pltpu.VMEM((N_DEV, 8, 128), jnp.float32),
            pltpu.SemaphoreType.DMA((2,)),
            pltpu.SemaphoreType.DMA((2,)),
            pltpu.SemaphoreType.REGULAR((2,)),
            pltpu.SemaphoreType.DMA((N_DEV - 1,)),
            pltpu.SemaphoreType.DMA((N_DEV,)),
        ],
        compiler_params=pltpu.CompilerParams(collective_id=0),
    )(x, w_mat)


# baseline (device time: 1423696 ns/iter reference)
import jax
import jax.numpy as jnp
from jax import lax
from jax.experimental import pallas as pl
from jax.experimental.pallas import tpu as pltpu

N_DEV = 16


def kernel(x, w_mat):
    m_total, k_per = x.shape
    _, n_cols = w_mat.shape
    m_per = m_total // N_DEV

    def body(x_ref, w_ref, out_ref,
             recv_buf, send_buf, maxes_ref,
             send_sems, recv_sems, credit_sems,
             b_send_sems, b_recv_sems):
        me = lax.axis_index("i")
        left = (me + N_DEV - 1) % N_DEV
        right = (me + 1) % N_DEV

        barrier_sem = pltpu.get_barrier_semaphore()
        for nbr in (left, right):
            pl.semaphore_signal(barrier_sem, inc=1, device_id=(nbr,),
                                device_id_type=pl.DeviceIdType.MESH)
        pl.semaphore_wait(barrier_sem, 2)

        def ring_desc(src_slot, dst_slot, sem_slot, target):
            return pltpu.make_async_remote_copy(
                src_ref=send_buf.at[src_slot],
                dst_ref=recv_buf.at[dst_slot],
                send_sem=send_sems.at[sem_slot],
                recv_sem=recv_sems.at[dst_slot],
                device_id=(target,),
                device_id_type=pl.DeviceIdType.MESH,
            )

        send_desc = [None, None]
        total = None

        for s in range(N_DEV):
            c = (me + 2 * N_DEV - 1 - s) % N_DEV
            k = s % 2
            xs = x_ref[pl.ds(c * m_per, m_per), :]
            p = jnp.dot(xs, w_ref[...], preferred_element_type=jnp.float32)

            if s == 0:
                send_buf[k, :, :] = p
            else:
                j = (s - 1) % 2
                ring_desc(j, j, j, left).wait_recv()
                total = recv_buf[j, :, :] + p
                if s < N_DEV - 1:
                    if send_desc[k] is not None:
                        send_desc[k].wait_send()
                        send_desc[k] = None
                    send_buf[k, :, :] = total
                if s <= N_DEV - 3:
                    pl.semaphore_signal(credit_sems.at[j], inc=1,
                                        device_id=(left,),
                                        device_id_type=pl.DeviceIdType.MESH)

            if s < N_DEV - 1:
                if s >= 2:
                    pl.semaphore_wait(credit_sems.at[k], 1)
                d = ring_desc(k, k, k, right)
                d.start()
                send_desc[k] = d

        for k in range(2):
            if send_desc[k] is not None:
                send_desc[k].wait_send()

        y = jnp.maximum(total, 0.0)
        out_ref[...] = y

        local_max = jnp.max(y)
        maxes_ref[pl.ds(me, 1)] = jnp.full((1, 8, 128), local_max,
                                           dtype=jnp.float32)

        bdescs = []
        for off in range(1, N_DEV):
            t = (me + off) % N_DEV
            bd = pltpu.make_async_remote_copy(
                src_ref=maxes_ref.at[me],
                dst_ref=maxes_ref.at[me],
                send_sem=b_send_sems.at[off - 1],
                recv_sem=b_recv_sems.at[me],
                device_id=(t,),
                device_id_type=pl.DeviceIdType.MESH,
            )
            bd.start()
            bdescs.append(bd)

        for j in range(N_DEV):
            @pl.when(j != me)
            def _(j=j):
                rd = pltpu.make_async_remote_copy(
                    src_ref=maxes_ref.at[j],
                    dst_ref=maxes_ref.at[j],
                    send_sem=b_send_sems.at[0],
                    recv_sem=b_recv_sems.at[j],
                    device_id=(left,),
                    device_id_type=pl.DeviceIdType.MESH,
                )
                rd.wait_recv()

        for bd in bdescs:
            bd.wait_send()

        gmax = jnp.max(maxes_ref[...])
        scale = jnp.maximum(gmax, 1e-30) / 127.0
        q = jnp.clip(jnp.round(out_ref[...] / scale), 0.0, 127.0)
        out_ref[...] = q * scale

    return pl.pallas_call(
        body,
        out_shape=jax.ShapeDtypeStruct((m_per, n_cols), jnp.float32),
        in_specs=[
            pl.BlockSpec(memory_space=pltpu.VMEM),
            pl.BlockSpec(memory_space=pltpu.VMEM),
        ],
        out_specs=pl.BlockSpec(memory_space=pltpu.VMEM),
        scratch_shapes=[
            pltpu.VMEM((2, m_per, n_cols), jnp.float32),
            pltpu.VMEM((2, m_per, n_cols), jnp.float32),
            pltpu.VMEM((N_DEV, 8, 128), jnp.float32),
            pltpu.SemaphoreType.DMA((2,)),
            pltpu.SemaphoreType.DMA((2,)),
            pltpu.SemaphoreType.REGULAR((2,)),
            pltpu.SemaphoreType.DMA((N_DEV - 1,)),
            pltpu.SemaphoreType.DMA((N_DEV,)),
        ],
        compiler_params=pltpu.CompilerParams(
            collective_id=0,
            vmem_limit_bytes=100 * 1024 * 1024,
        ),
    )(x, w_mat)
